# baseline (speedup 1.0000x reference)
import jax
import jax.numpy as jnp
from jax import lax
from jax.experimental import pallas as pl
from jax.experimental.pallas import tpu as pltpu
from jax.experimental.pallas import tpu_sc as plsc

_RES = 256
_B = 1048576
_NC = 2
_NS = 16
_NW = _NC * _NS
_PW = _B // _NW
_CHUNK = 128
_NCHUNK = _PW // _CHUNK
_SUPER = 4096
_GPS = _SUPER // _CHUNK


def _quad_plane(dp, cp):
    t = jnp.transpose(jnp.concatenate([dp, cp], axis=0), (1, 2, 0))
    ip = jnp.minimum(jnp.arange(_RES) + 1, _RES - 1)
    t01 = t[:, ip, :]
    t10 = t[ip, :, :]
    t11 = t10[:, ip, :]
    return jnp.stack([t, t01, t10, t11], axis=2).reshape(_RES * _RES, 64)


def _pair_line(dv, cv):
    t = jnp.concatenate([dv, cv], axis=0)
    ip = jnp.minimum(jnp.arange(_RES) + 1, _RES - 1)
    return jnp.concatenate([t, t[:, ip]], axis=0)


def _axis_weights(p):
    i0 = jnp.minimum(jnp.maximum(p.astype(jnp.int32), 0), _RES - 1)
    f0 = i0.astype(jnp.float32)
    i1 = jnp.minimum(i0 + 1, _RES - 1)
    f1 = i1.astype(jnp.float32)
    return i0, f1 - p, p - f0


def _sc_body(xs, ys, zs, qyz, qzx, qxy, lx, ly, lz, feat,
             xb, yb, zb,
             ppx, ppy, ppz,
             iyz, izx, ixy,
             gyz, gzx, gxy,
             ltx, lty, ltz,
             fv, gsem, osem):
    wid = lax.axis_index("s") * _NC + lax.axis_index("c")
    base = wid * _PW
    pltpu.sync_copy(lx, ltx)
    pltpu.sync_copy(ly, lty)
    pltpu.sync_copy(lz, ltz)

    def issue(c, k):
        soff = lax.rem(c, _GPS) * _CHUNK

        def phase_a(j, carry):
            s = pl.ds(soff + j * 16, 16)
            d = pl.ds(j * 16, 16)
            px = (xb[s] + 1.0) * 127.5
            py = (yb[s] + 1.0) * 127.5
            pz = (zb[s] + 1.0) * 127.5
            x0 = jnp.minimum(jnp.maximum(px.astype(jnp.int32), 0), _RES - 1)
            y0 = jnp.minimum(jnp.maximum(py.astype(jnp.int32), 0), _RES - 1)
            z0 = jnp.minimum(jnp.maximum(pz.astype(jnp.int32), 0), _RES - 1)
            ppx[k, d] = px
            ppy[k, d] = py
            ppz[k, d] = pz
            iyz[k, d] = y0 * _RES + z0
            izx[k, d] = z0 * _RES + x0
            ixy[k, d] = x0 * _RES + y0
            return carry

        lax.fori_loop(0, _CHUNK // 16, phase_a, 0, unroll=2)
        pltpu.async_copy(qyz.at[iyz.at[k]], gyz.at[k], gsem.at[k])
        pltpu.async_copy(qzx.at[izx.at[k]], gzx.at[k], gsem.at[k])
        pltpu.async_copy(qxy.at[ixy.at[k]], gxy.at[k], gsem.at[k])

    def drain_gathers(k):
        pltpu.make_async_copy(qyz.at[iyz.at[k]], gyz.at[k], gsem.at[k]).wait()
        pltpu.make_async_copy(qzx.at[izx.at[k]], gzx.at[k], gsem.at[k]).wait()
        pltpu.make_async_copy(qxy.at[ixy.at[k]], gxy.at[k], gsem.at[k]).wait()

    def combine(c, k):
        off = base + c * _CHUNK

        @pl.when(c >= 2)
        def _():
            pltpu.make_async_copy(fv.at[k], feat.at[pl.ds(off, _CHUNK)],
                                  osem.at[k]).wait()

        kv = jnp.full((16,), k, jnp.int32)
        lane = jnp.arange(16, dtype=jnp.int32)
        lane_hi = lane + 16

        def phase_b(i, carry):
            ii = jnp.full((16,), i, jnp.int32)
            x0, sax, sbx = _axis_weights(plsc.load_gather(ppx, [kv, ii]))
            y0, say, sby = _axis_weights(plsc.load_gather(ppy, [kv, ii]))
            z0, saz, sbz = _axis_weights(plsc.load_gather(ppz, [kv, ii]))

            p0 = ((gyz[k, i, pl.ds(0, 16)] * saz + gyz[k, i, pl.ds(16, 16)] * sbz) * say
                  + (gyz[k, i, pl.ds(32, 16)] * saz + gyz[k, i, pl.ds(48, 16)] * sbz) * sby)
            l0 = (plsc.load_gather(ltx, [lane, x0]) * sax
                  + plsc.load_gather(ltx, [lane_hi, x0]) * sbx)
            fv[k, i, pl.ds(0, 16)] = l0 * p0

            p1 = ((gzx[k, i, pl.ds(0, 16)] * sax + gzx[k, i, pl.ds(16, 16)] * sbx) * saz
                  + (gzx[k, i, pl.ds(32, 16)] * sax + gzx[k, i, pl.ds(48, 16)] * sbx) * sbz)
            l1 = (plsc.load_gather(lty, [lane, y0]) * say
                  + plsc.load_gather(lty, [lane_hi, y0]) * sby)
            fv[k, i, pl.ds(16, 16)] = l1 * p1

            p2 = ((gxy[k, i, pl.ds(0, 16)] * say + gxy[k, i, pl.ds(16, 16)] * sby) * sax
                  + (gxy[k, i, pl.ds(32, 16)] * say + gxy[k, i, pl.ds(48, 16)] * sby) * sbx)
            l2 = (plsc.load_gather(ltz, [lane, z0]) * saz
                  + plsc.load_gather(ltz, [lane_hi, z0]) * sbz)
            fv[k, i, pl.ds(32, 16)] = l2 * p2
            return carry

        lax.fori_loop(0, _CHUNK, phase_b, 0, unroll=2)
        pltpu.async_copy(fv.at[k], feat.at[pl.ds(off, _CHUNK)], osem.at[k])

    def step(c, carry):
        k = jnp.bitwise_and(c, 1)

        @pl.when(lax.rem(c, _GPS) == 0)
        def _():
            goff = base + c * _CHUNK
            pltpu.sync_copy(xs.at[pl.ds(goff, _SUPER)], xb)
            pltpu.sync_copy(ys.at[pl.ds(goff, _SUPER)], yb)
            pltpu.sync_copy(zs.at[pl.ds(goff, _SUPER)], zb)

        issue(c, k)

        @pl.when(c > 0)
        def _():
            drain_gathers(1 - k)
            combine(c - 1, 1 - k)

        return carry

    lax.fori_loop(0, _NCHUNK, step, 0)
    last = _NCHUNK - 1
    drain_gathers(last % 2)
    combine(last, last % 2)
    pltpu.make_async_copy(fv.at[0], feat.at[pl.ds(base, _CHUNK)], osem.at[0]).wait()
    pltpu.make_async_copy(fv.at[1], feat.at[pl.ds(base, _CHUNK)], osem.at[1]).wait()


def _mm_body(f_ref, w_ref, o_ref):
    o_ref[...] = jnp.dot(f_ref[...], w_ref[...],
                         preferred_element_type=jnp.float32)


def kernel(xyz, dvx, dvy, dvz, dpyz, dpzx, dpxy, cvx, cvy, cvz, cpyz, cpzx, cpxy, W):
    qyz = _quad_plane(dpyz, cpyz)
    qzx = _quad_plane(dpzx, cpzx)
    qxy = _quad_plane(dpxy, cpxy)
    lx = _pair_line(dvx, cvx)
    ly = _pair_line(dvy, cvy)
    lz = _pair_line(dvz, cvz)
    xs = xyz[:, 0]
    ys = xyz[:, 1]
    zs = xyz[:, 2]

    mesh = plsc.VectorSubcoreMesh(core_axis_name="c", subcore_axis_name="s")
    f32 = jnp.float32
    i32 = jnp.int32
    sc = pl.kernel(
        _sc_body,
        out_type=jax.ShapeDtypeStruct((_B, 48), f32),
        mesh=mesh,
        compiler_params=pltpu.CompilerParams(
            needs_layout_passes=False, use_tc_tiling_on_sc=False),
        scratch_types=[
            pltpu.VMEM((_SUPER,), f32), pltpu.VMEM((_SUPER,), f32), pltpu.VMEM((_SUPER,), f32),
            pltpu.VMEM((2, _CHUNK), f32), pltpu.VMEM((2, _CHUNK), f32), pltpu.VMEM((2, _CHUNK), f32),
            pltpu.VMEM((2, _CHUNK), i32), pltpu.VMEM((2, _CHUNK), i32), pltpu.VMEM((2, _CHUNK), i32),
            pltpu.VMEM((2, _CHUNK, 64), f32), pltpu.VMEM((2, _CHUNK, 64), f32), pltpu.VMEM((2, _CHUNK, 64), f32),
            pltpu.VMEM((32, _RES), f32), pltpu.VMEM((32, _RES), f32), pltpu.VMEM((32, _RES), f32),
            pltpu.VMEM((2, _CHUNK, 48), f32),
            pltpu.SemaphoreType.DMA((2,)),
            pltpu.SemaphoreType.DMA((2,)),
        ],
    )
    feat = sc(xs, ys, zs, qyz, qzx, qxy, lx, ly, lz)

    wb = jnp.zeros((48, 28), f32)
    dense_rows = jnp.concatenate(
        [jnp.arange(8), jnp.arange(8) + 16, jnp.arange(8) + 32])
    color_rows = jnp.concatenate(
        [jnp.arange(8) + 8, jnp.arange(8) + 24, jnp.arange(8) + 40])
    wb = wb.at[dense_rows, 0].set(1.0)
    wb = wb.at[color_rows, 1:28].set(W.T)

    tm = 2048
    out = pl.pallas_call(
        _mm_body,
        grid=(_B // tm,),
        in_specs=[
            pl.BlockSpec((tm, 48), lambda i: (i, 0)),
            pl.BlockSpec((48, 28), lambda i: (0, 0)),
        ],
        out_specs=pl.BlockSpec((tm, 28), lambda i: (i, 0)),
        out_shape=jax.ShapeDtypeStruct((_B, 28), f32),
    )(feat, wb)
    return out


# speedup vs baseline: 1.6663x; 1.6663x over previous
import jax
import jax.numpy as jnp
from jax import lax
from jax.experimental import pallas as pl
from jax.experimental.pallas import tpu as pltpu
from jax.experimental.pallas import tpu_sc as plsc

_RES = 256
_B = 1048576
_BH = _B // 2
_NC = 2
_NS = 16
_NW = _NC * _NS
_PW = _BH // _NW
_CHUNK = 128
_NCHUNK = _PW // _CHUNK
_SUPER = 4096
_GPS = _SUPER // _CHUNK


def _quad_plane(dp, cp):
    t = jnp.transpose(jnp.concatenate([dp, cp], axis=0), (1, 2, 0))
    ip = jnp.minimum(jnp.arange(_RES) + 1, _RES - 1)
    t01 = t[:, ip, :]
    t10 = t[ip, :, :]
    t11 = t10[:, ip, :]
    return jnp.stack([t, t01, t10, t11], axis=2).reshape(_RES * _RES, 64)


def _pair_line(dv, cv):
    t = jnp.concatenate([dv, cv], axis=0).T
    ip = jnp.minimum(jnp.arange(_RES) + 1, _RES - 1)
    return jnp.concatenate([t, t[ip]], axis=1)


def _axis_weights(p):
    i0 = jnp.minimum(jnp.maximum(p.astype(jnp.int32), 0), _RES - 1)
    f0 = i0.astype(jnp.float32)
    i1 = jnp.minimum(i0 + 1, _RES - 1)
    f1 = i1.astype(jnp.float32)
    return i0, f1 - p, p - f0


def _sc_body(xs, ys, zs, qyz, qzx, qxy, lx, ly, lz, feat,
             xb, yb, zb,
             ppx, ppy, ppz,
             iyz, izx, ixy, ilx, ily, ilz,
             gyz, gzx, gxy, glx, gly, glz,
             fv, gsem, osem):
    wid = lax.axis_index("s") * _NC + lax.axis_index("c")
    base = wid * _PW

    def issue(c, k):
        soff = lax.rem(c, _GPS) * _CHUNK

        def phase_a(j, carry):
            s = pl.ds(soff + j * 16, 16)
            d = pl.ds(j * 16, 16)
            px = (xb[s] + 1.0) * 127.5
            py = (yb[s] + 1.0) * 127.5
            pz = (zb[s] + 1.0) * 127.5
            x0 = jnp.minimum(jnp.maximum(px.astype(jnp.int32), 0), _RES - 1)
            y0 = jnp.minimum(jnp.maximum(py.astype(jnp.int32), 0), _RES - 1)
            z0 = jnp.minimum(jnp.maximum(pz.astype(jnp.int32), 0), _RES - 1)
            ppx[k, d] = px
            ppy[k, d] = py
            ppz[k, d] = pz
            iyz[k, d] = y0 * _RES + z0
            izx[k, d] = z0 * _RES + x0
            ixy[k, d] = x0 * _RES + y0
            ilx[k, d] = x0
            ily[k, d] = y0
            ilz[k, d] = z0
            return carry

        lax.fori_loop(0, _CHUNK // 16, phase_a, 0, unroll=2)
        pltpu.async_copy(qyz.at[iyz.at[k]], gyz.at[k], gsem.at[k])
        pltpu.async_copy(qzx.at[izx.at[k]], gzx.at[k], gsem.at[k])
        pltpu.async_copy(qxy.at[ixy.at[k]], gxy.at[k], gsem.at[k])
        pltpu.async_copy(lx.at[ilx.at[k]], glx.at[k], gsem.at[k])
        pltpu.async_copy(ly.at[ily.at[k]], gly.at[k], gsem.at[k])
        pltpu.async_copy(lz.at[ilz.at[k]], glz.at[k], gsem.at[k])

    def drain_gathers(k):
        pltpu.make_async_copy(qyz.at[iyz.at[k]], gyz.at[k], gsem.at[k]).wait()
        pltpu.make_async_copy(qzx.at[izx.at[k]], gzx.at[k], gsem.at[k]).wait()
        pltpu.make_async_copy(qxy.at[ixy.at[k]], gxy.at[k], gsem.at[k]).wait()
        pltpu.make_async_copy(lx.at[ilx.at[k]], glx.at[k], gsem.at[k]).wait()
        pltpu.make_async_copy(ly.at[ily.at[k]], gly.at[k], gsem.at[k]).wait()
        pltpu.make_async_copy(lz.at[ilz.at[k]], glz.at[k], gsem.at[k]).wait()

    def combine(c, k):
        off = base + c * _CHUNK

        @pl.when(c >= 2)
        def _():
            pltpu.make_async_copy(fv.at[k], feat.at[pl.ds(off, _CHUNK)],
                                  osem.at[k]).wait()

        kv = jnp.full((16,), k, jnp.int32)

        def phase_b(i, carry):
            ii = jnp.full((16,), i, jnp.int32)
            _, sax, sbx = _axis_weights(plsc.load_gather(ppx, [kv, ii]))
            _, say, sby = _axis_weights(plsc.load_gather(ppy, [kv, ii]))
            _, saz, sbz = _axis_weights(plsc.load_gather(ppz, [kv, ii]))

            p0 = ((gyz[k, i, pl.ds(0, 16)] * saz + gyz[k, i, pl.ds(16, 16)] * sbz) * say
                  + (gyz[k, i, pl.ds(32, 16)] * saz + gyz[k, i, pl.ds(48, 16)] * sbz) * sby)
            l0 = glx[k, i, pl.ds(0, 16)] * sax + glx[k, i, pl.ds(16, 16)] * sbx
            fv[k, i, pl.ds(0, 16)] = l0 * p0

            p1 = ((gzx[k, i, pl.ds(0, 16)] * sax + gzx[k, i, pl.ds(16, 16)] * sbx) * saz
                  + (gzx[k, i, pl.ds(32, 16)] * sax + gzx[k, i, pl.ds(48, 16)] * sbx) * sbz)
            l1 = gly[k, i, pl.ds(0, 16)] * say + gly[k, i, pl.ds(16, 16)] * sby
            fv[k, i, pl.ds(16, 16)] = l1 * p1

            p2 = ((gxy[k, i, pl.ds(0, 16)] * say + gxy[k, i, pl.ds(16, 16)] * sby) * sax
                  + (gxy[k, i, pl.ds(32, 16)] * say + gxy[k, i, pl.ds(48, 16)] * sby) * sbx)
            l2 = glz[k, i, pl.ds(0, 16)] * saz + glz[k, i, pl.ds(16, 16)] * sbz
            fv[k, i, pl.ds(32, 16)] = l2 * p2
            return carry

        lax.fori_loop(0, _CHUNK, phase_b, 0, unroll=2)
        pltpu.async_copy(fv.at[k], feat.at[pl.ds(off, _CHUNK)], osem.at[k])

    def step(c, carry):
        k = jnp.bitwise_and(c, 1)

        @pl.when(lax.rem(c, _GPS) == 0)
        def _():
            goff = base + c * _CHUNK
            pltpu.sync_copy(xs.at[pl.ds(goff, _SUPER)], xb)
            pltpu.sync_copy(ys.at[pl.ds(goff, _SUPER)], yb)
            pltpu.sync_copy(zs.at[pl.ds(goff, _SUPER)], zb)

        issue(c, k)

        @pl.when(c > 0)
        def _():
            drain_gathers(1 - k)
            combine(c - 1, 1 - k)

        return carry

    lax.fori_loop(0, _NCHUNK, step, 0)
    last = _NCHUNK - 1
    drain_gathers(last % 2)
    combine(last, last % 2)
    pltpu.make_async_copy(fv.at[0], feat.at[pl.ds(base, _CHUNK)], osem.at[0]).wait()
    pltpu.make_async_copy(fv.at[1], feat.at[pl.ds(base, _CHUNK)], osem.at[1]).wait()


def _mm_body(f_ref, w_ref, o_ref):
    o_ref[...] = jnp.dot(f_ref[...], w_ref[...],
                         preferred_element_type=jnp.float32)


def kernel(xyz, dvx, dvy, dvz, dpyz, dpzx, dpxy, cvx, cvy, cvz, cpyz, cpzx, cpxy, W):
    qyz = _quad_plane(dpyz, cpyz)
    qzx = _quad_plane(dpzx, cpzx)
    qxy = _quad_plane(dpxy, cpxy)
    lx = _pair_line(dvx, cvx)
    ly = _pair_line(dvy, cvy)
    lz = _pair_line(dvz, cvz)
    xs = xyz[:, 0]
    ys = xyz[:, 1]
    zs = xyz[:, 2]

    mesh = plsc.VectorSubcoreMesh(core_axis_name="c", subcore_axis_name="s")
    f32 = jnp.float32
    i32 = jnp.int32
    sc = pl.kernel(
        _sc_body,
        out_type=jax.ShapeDtypeStruct((_BH, 48), f32),
        mesh=mesh,
        compiler_params=pltpu.CompilerParams(
            needs_layout_passes=False, use_tc_tiling_on_sc=False),
        scratch_types=[
            pltpu.VMEM((_SUPER,), f32), pltpu.VMEM((_SUPER,), f32), pltpu.VMEM((_SUPER,), f32),
            pltpu.VMEM((2, _CHUNK), f32), pltpu.VMEM((2, _CHUNK), f32), pltpu.VMEM((2, _CHUNK), f32),
            pltpu.VMEM((2, _CHUNK), i32), pltpu.VMEM((2, _CHUNK), i32), pltpu.VMEM((2, _CHUNK), i32),
            pltpu.VMEM((2, _CHUNK), i32), pltpu.VMEM((2, _CHUNK), i32), pltpu.VMEM((2, _CHUNK), i32),
            pltpu.VMEM((2, _CHUNK, 64), f32), pltpu.VMEM((2, _CHUNK, 64), f32), pltpu.VMEM((2, _CHUNK, 64), f32),
            pltpu.VMEM((2, _CHUNK, 32), f32), pltpu.VMEM((2, _CHUNK, 32), f32), pltpu.VMEM((2, _CHUNK, 32), f32),
            pltpu.VMEM((2, _CHUNK, 48), f32),
            pltpu.SemaphoreType.DMA((2,)),
            pltpu.SemaphoreType.DMA((2,)),
        ],
    )
    wb = jnp.zeros((48, 28), f32)
    dense_rows = jnp.concatenate(
        [jnp.arange(8), jnp.arange(8) + 16, jnp.arange(8) + 32])
    color_rows = jnp.concatenate(
        [jnp.arange(8) + 8, jnp.arange(8) + 24, jnp.arange(8) + 40])
    wb = wb.at[dense_rows, 0].set(1.0)
    wb = wb.at[color_rows, 1:28].set(W.T)

    tm = 2048
    mm = pl.pallas_call(
        _mm_body,
        grid=(_BH // tm,),
        in_specs=[
            pl.BlockSpec((tm, 48), lambda i: (i, 0)),
            pl.BlockSpec((48, 28), lambda i: (0, 0)),
        ],
        out_specs=pl.BlockSpec((tm, 28), lambda i: (i, 0)),
        out_shape=jax.ShapeDtypeStruct((_BH, 28), f32),
    )

    halves = []
    for h in range(2):
        s = slice(h * _BH, (h + 1) * _BH)
        feat = sc(xs[s], ys[s], zs[s], qyz, qzx, qxy, lx, ly, lz)
        halves.append(mm(feat, wb))
    return jnp.concatenate(halves, axis=0)


# speedup vs baseline: 1.7423x; 1.0456x over previous
import jax
import jax.numpy as jnp
from jax import lax
from jax.experimental import pallas as pl
from jax.experimental.pallas import tpu as pltpu
from jax.experimental.pallas import tpu_sc as plsc

_RES = 256
_B = 1048576
_NSPLIT = 4
_BH = _B // _NSPLIT
_NC = 2
_NS = 16
_NW = _NC * _NS
_PW = _BH // _NW
_CHUNK = 128
_NCHUNK = _PW // _CHUNK
_SUPER = 4096
_GPS = _SUPER // _CHUNK


def _quad_plane(dp, cp):
    t = jnp.transpose(jnp.concatenate([dp, cp], axis=0), (1, 2, 0))
    ip = jnp.minimum(jnp.arange(_RES) + 1, _RES - 1)
    t01 = t[:, ip, :]
    t10 = t[ip, :, :]
    t11 = t10[:, ip, :]
    return jnp.stack([t, t01, t10, t11], axis=2).reshape(_RES * _RES, 64)


def _pair_line(dv, cv):
    t = jnp.concatenate([dv, cv], axis=0).T
    ip = jnp.minimum(jnp.arange(_RES) + 1, _RES - 1)
    return jnp.concatenate([t, t[ip]], axis=1)


def _axis_weights(p):
    i0 = jnp.minimum(jnp.maximum(p.astype(jnp.int32), 0), _RES - 1)
    f0 = i0.astype(jnp.float32)
    i1 = jnp.minimum(i0 + 1, _RES - 1)
    f1 = i1.astype(jnp.float32)
    return i0, f1 - p, p - f0


def _sc_body(xs, ys, zs, qyz, qzx, qxy, lx, ly, lz, feat,
             xb, yb, zb,
             ppx, ppy, ppz,
             iyz, izx, ixy, ilx, ily, ilz,
             gyz, gzx, gxy, glx, gly, glz,
             fv, gsem, osem):
    wid = lax.axis_index("s") * _NC + lax.axis_index("c")
    base = wid * _PW

    def issue(c, k):
        soff = lax.rem(c, _GPS) * _CHUNK

        def phase_a(j, carry):
            s = pl.ds(soff + j * 16, 16)
            d = pl.ds(j * 16, 16)
            px = (xb[s] + 1.0) * 127.5
            py = (yb[s] + 1.0) * 127.5
            pz = (zb[s] + 1.0) * 127.5
            x0 = jnp.minimum(jnp.maximum(px.astype(jnp.int32), 0), _RES - 1)
            y0 = jnp.minimum(jnp.maximum(py.astype(jnp.int32), 0), _RES - 1)
            z0 = jnp.minimum(jnp.maximum(pz.astype(jnp.int32), 0), _RES - 1)
            ppx[k, d] = px
            ppy[k, d] = py
            ppz[k, d] = pz
            iyz[k, d] = y0 * _RES + z0
            izx[k, d] = z0 * _RES + x0
            ixy[k, d] = x0 * _RES + y0
            ilx[k, d] = x0
            ily[k, d] = y0
            ilz[k, d] = z0
            return carry

        lax.fori_loop(0, _CHUNK // 16, phase_a, 0, unroll=2)
        pltpu.async_copy(qyz.at[iyz.at[k]], gyz.at[k], gsem.at[k])
        pltpu.async_copy(qzx.at[izx.at[k]], gzx.at[k], gsem.at[k])
        pltpu.async_copy(qxy.at[ixy.at[k]], gxy.at[k], gsem.at[k])
        pltpu.async_copy(lx.at[ilx.at[k]], glx.at[k], gsem.at[k])
        pltpu.async_copy(ly.at[ily.at[k]], gly.at[k], gsem.at[k])
        pltpu.async_copy(lz.at[ilz.at[k]], glz.at[k], gsem.at[k])

    def drain_gathers(k):
        pltpu.make_async_copy(qyz.at[iyz.at[k]], gyz.at[k], gsem.at[k]).wait()
        pltpu.make_async_copy(qzx.at[izx.at[k]], gzx.at[k], gsem.at[k]).wait()
        pltpu.make_async_copy(qxy.at[ixy.at[k]], gxy.at[k], gsem.at[k]).wait()
        pltpu.make_async_copy(lx.at[ilx.at[k]], glx.at[k], gsem.at[k]).wait()
        pltpu.make_async_copy(ly.at[ily.at[k]], gly.at[k], gsem.at[k]).wait()
        pltpu.make_async_copy(lz.at[ilz.at[k]], glz.at[k], gsem.at[k]).wait()

    def combine(c, k):
        off = base + c * _CHUNK

        @pl.when(c >= 2)
        def _():
            pltpu.make_async_copy(fv.at[k], feat.at[pl.ds(off, _CHUNK)],
                                  osem.at[k]).wait()

        kv = jnp.full((16,), k, jnp.int32)

        def phase_b(i, carry):
            ii = jnp.full((16,), i, jnp.int32)
            _, sax, sbx = _axis_weights(plsc.load_gather(ppx, [kv, ii]))
            _, say, sby = _axis_weights(plsc.load_gather(ppy, [kv, ii]))
            _, saz, sbz = _axis_weights(plsc.load_gather(ppz, [kv, ii]))

            p0 = ((gyz[k, i, pl.ds(0, 16)] * saz + gyz[k, i, pl.ds(16, 16)] * sbz) * say
                  + (gyz[k, i, pl.ds(32, 16)] * saz + gyz[k, i, pl.ds(48, 16)] * sbz) * sby)
            l0 = glx[k, i, pl.ds(0, 16)] * sax + glx[k, i, pl.ds(16, 16)] * sbx
            fv[k, i, pl.ds(0, 16)] = l0 * p0

            p1 = ((gzx[k, i, pl.ds(0, 16)] * sax + gzx[k, i, pl.ds(16, 16)] * sbx) * saz
                  + (gzx[k, i, pl.ds(32, 16)] * sax + gzx[k, i, pl.ds(48, 16)] * sbx) * sbz)
            l1 = gly[k, i, pl.ds(0, 16)] * say + gly[k, i, pl.ds(16, 16)] * sby
            fv[k, i, pl.ds(16, 16)] = l1 * p1

            p2 = ((gxy[k, i, pl.ds(0, 16)] * say + gxy[k, i, pl.ds(16, 16)] * sby) * sax
                  + (gxy[k, i, pl.ds(32, 16)] * say + gxy[k, i, pl.ds(48, 16)] * sby) * sbx)
            l2 = glz[k, i, pl.ds(0, 16)] * saz + glz[k, i, pl.ds(16, 16)] * sbz
            fv[k, i, pl.ds(32, 16)] = l2 * p2
            return carry

        lax.fori_loop(0, _CHUNK, phase_b, 0, unroll=2)
        pltpu.async_copy(fv.at[k], feat.at[pl.ds(off, _CHUNK)], osem.at[k])

    def step(c, carry):
        k = jnp.bitwise_and(c, 1)

        @pl.when(lax.rem(c, _GPS) == 0)
        def _():
            goff = base + c * _CHUNK
            pltpu.sync_copy(xs.at[pl.ds(goff, _SUPER)], xb)
            pltpu.sync_copy(ys.at[pl.ds(goff, _SUPER)], yb)
            pltpu.sync_copy(zs.at[pl.ds(goff, _SUPER)], zb)

        issue(c, k)

        @pl.when(c > 0)
        def _():
            drain_gathers(1 - k)
            combine(c - 1, 1 - k)

        return carry

    lax.fori_loop(0, _NCHUNK, step, 0)
    last = _NCHUNK - 1
    drain_gathers(last % 2)
    combine(last, last % 2)
    pltpu.make_async_copy(fv.at[0], feat.at[pl.ds(base, _CHUNK)], osem.at[0]).wait()
    pltpu.make_async_copy(fv.at[1], feat.at[pl.ds(base, _CHUNK)], osem.at[1]).wait()


def _mm_body(f_ref, w_ref, o_ref):
    o_ref[...] = jnp.dot(f_ref[...], w_ref[...],
                         preferred_element_type=jnp.float32)


def kernel(xyz, dvx, dvy, dvz, dpyz, dpzx, dpxy, cvx, cvy, cvz, cpyz, cpzx, cpxy, W):
    qyz = _quad_plane(dpyz, cpyz)
    qzx = _quad_plane(dpzx, cpzx)
    qxy = _quad_plane(dpxy, cpxy)
    lx = _pair_line(dvx, cvx)
    ly = _pair_line(dvy, cvy)
    lz = _pair_line(dvz, cvz)
    xs = xyz[:, 0]
    ys = xyz[:, 1]
    zs = xyz[:, 2]

    mesh = plsc.VectorSubcoreMesh(core_axis_name="c", subcore_axis_name="s")
    f32 = jnp.float32
    i32 = jnp.int32
    sc = pl.kernel(
        _sc_body,
        out_type=jax.ShapeDtypeStruct((_BH, 48), f32),
        mesh=mesh,
        compiler_params=pltpu.CompilerParams(
            needs_layout_passes=False, use_tc_tiling_on_sc=False),
        scratch_types=[
            pltpu.VMEM((_SUPER,), f32), pltpu.VMEM((_SUPER,), f32), pltpu.VMEM((_SUPER,), f32),
            pltpu.VMEM((2, _CHUNK), f32), pltpu.VMEM((2, _CHUNK), f32), pltpu.VMEM((2, _CHUNK), f32),
            pltpu.VMEM((2, _CHUNK), i32), pltpu.VMEM((2, _CHUNK), i32), pltpu.VMEM((2, _CHUNK), i32),
            pltpu.VMEM((2, _CHUNK), i32), pltpu.VMEM((2, _CHUNK), i32), pltpu.VMEM((2, _CHUNK), i32),
            pltpu.VMEM((2, _CHUNK, 64), f32), pltpu.VMEM((2, _CHUNK, 64), f32), pltpu.VMEM((2, _CHUNK, 64), f32),
            pltpu.VMEM((2, _CHUNK, 32), f32), pltpu.VMEM((2, _CHUNK, 32), f32), pltpu.VMEM((2, _CHUNK, 32), f32),
            pltpu.VMEM((2, _CHUNK, 48), f32),
            pltpu.SemaphoreType.DMA((2,)),
            pltpu.SemaphoreType.DMA((2,)),
        ],
    )
    wb = jnp.zeros((48, 28), f32)
    dense_rows = jnp.concatenate(
        [jnp.arange(8), jnp.arange(8) + 16, jnp.arange(8) + 32])
    color_rows = jnp.concatenate(
        [jnp.arange(8) + 8, jnp.arange(8) + 24, jnp.arange(8) + 40])
    wb = wb.at[dense_rows, 0].set(1.0)
    wb = wb.at[color_rows, 1:28].set(W.T)

    tm = 2048
    mm = pl.pallas_call(
        _mm_body,
        grid=(_BH // tm,),
        in_specs=[
            pl.BlockSpec((tm, 48), lambda i: (i, 0)),
            pl.BlockSpec((48, 28), lambda i: (0, 0)),
        ],
        out_specs=pl.BlockSpec((tm, 28), lambda i: (i, 0)),
        out_shape=jax.ShapeDtypeStruct((_BH, 28), f32),
    )

    parts = []
    for h in range(_NSPLIT):
        s = slice(h * _BH, (h + 1) * _BH)
        feat = sc(xs[s], ys[s], zs[s], qyz, qzx, qxy, lx, ly, lz)
        parts.append(mm(feat, wb))
    return jnp.concatenate(parts, axis=0)


# speedup vs baseline: 1.8054x; 1.0363x over previous
import jax
import jax.numpy as jnp
from jax import lax
from jax.experimental import pallas as pl
from jax.experimental.pallas import tpu as pltpu
from jax.experimental.pallas import tpu_sc as plsc

_RES = 256
_B = 1048576
_NSPLIT = 8
_BH = _B // _NSPLIT
_NC = 2
_NS = 16
_NW = _NC * _NS
_PW = _BH // _NW
_CHUNK = 128
_NCHUNK = _PW // _CHUNK
_SUPER = 4096
_GPS = _SUPER // _CHUNK


def _quad_plane(dp, cp):
    t = jnp.transpose(jnp.concatenate([dp, cp], axis=0), (1, 2, 0))
    ip = jnp.minimum(jnp.arange(_RES) + 1, _RES - 1)
    t01 = t[:, ip, :]
    t10 = t[ip, :, :]
    t11 = t10[:, ip, :]
    return jnp.stack([t, t01, t10, t11], axis=2).reshape(_RES * _RES, 64)


def _pair_line(dv, cv):
    t = jnp.concatenate([dv, cv], axis=0).T
    ip = jnp.minimum(jnp.arange(_RES) + 1, _RES - 1)
    return jnp.concatenate([t, t[ip]], axis=1)


def _axis_weights(p):
    i0 = jnp.minimum(jnp.maximum(p.astype(jnp.int32), 0), _RES - 1)
    f0 = i0.astype(jnp.float32)
    i1 = jnp.minimum(i0 + 1, _RES - 1)
    f1 = i1.astype(jnp.float32)
    return i0, f1 - p, p - f0


def _sc_body(xs, ys, zs, qyz, qzx, qxy, lx, ly, lz, feat,
             xb, yb, zb,
             ppx, ppy, ppz,
             iyz, izx, ixy, ilx, ily, ilz,
             gyz, gzx, gxy, glx, gly, glz,
             fv, gsem, osem):
    wid = lax.axis_index("s") * _NC + lax.axis_index("c")
    base = wid * _PW

    def issue(c, k):
        soff = lax.rem(c, _GPS) * _CHUNK

        def phase_a(j, carry):
            s = pl.ds(soff + j * 16, 16)
            d = pl.ds(j * 16, 16)
            px = (xb[s] + 1.0) * 127.5
            py = (yb[s] + 1.0) * 127.5
            pz = (zb[s] + 1.0) * 127.5
            x0 = jnp.minimum(jnp.maximum(px.astype(jnp.int32), 0), _RES - 1)
            y0 = jnp.minimum(jnp.maximum(py.astype(jnp.int32), 0), _RES - 1)
            z0 = jnp.minimum(jnp.maximum(pz.astype(jnp.int32), 0), _RES - 1)
            ppx[k, d] = px
            ppy[k, d] = py
            ppz[k, d] = pz
            iyz[k, d] = y0 * _RES + z0
            izx[k, d] = z0 * _RES + x0
            ixy[k, d] = x0 * _RES + y0
            ilx[k, d] = x0
            ily[k, d] = y0
            ilz[k, d] = z0
            return carry

        lax.fori_loop(0, _CHUNK // 16, phase_a, 0, unroll=2)
        pltpu.async_copy(qyz.at[iyz.at[k]], gyz.at[k], gsem.at[k])
        pltpu.async_copy(qzx.at[izx.at[k]], gzx.at[k], gsem.at[k])
        pltpu.async_copy(qxy.at[ixy.at[k]], gxy.at[k], gsem.at[k])
        pltpu.async_copy(lx.at[ilx.at[k]], glx.at[k], gsem.at[k])
        pltpu.async_copy(ly.at[ily.at[k]], gly.at[k], gsem.at[k])
        pltpu.async_copy(lz.at[ilz.at[k]], glz.at[k], gsem.at[k])

    def drain_gathers(k):
        pltpu.make_async_copy(qyz.at[iyz.at[k]], gyz.at[k], gsem.at[k]).wait()
        pltpu.make_async_copy(qzx.at[izx.at[k]], gzx.at[k], gsem.at[k]).wait()
        pltpu.make_async_copy(qxy.at[ixy.at[k]], gxy.at[k], gsem.at[k]).wait()
        pltpu.make_async_copy(lx.at[ilx.at[k]], glx.at[k], gsem.at[k]).wait()
        pltpu.make_async_copy(ly.at[ily.at[k]], gly.at[k], gsem.at[k]).wait()
        pltpu.make_async_copy(lz.at[ilz.at[k]], glz.at[k], gsem.at[k]).wait()

    def combine(c, k):
        off = base + c * _CHUNK

        @pl.when(c >= 2)
        def _():
            pltpu.make_async_copy(fv.at[k], feat.at[pl.ds(off, _CHUNK)],
                                  osem.at[k]).wait()

        kv = jnp.full((16,), k, jnp.int32)

        def phase_b(i, carry):
            ii = jnp.full((16,), i, jnp.int32)
            _, sax, sbx = _axis_weights(plsc.load_gather(ppx, [kv, ii]))
            _, say, sby = _axis_weights(plsc.load_gather(ppy, [kv, ii]))
            _, saz, sbz = _axis_weights(plsc.load_gather(ppz, [kv, ii]))

            p0 = ((gyz[k, i, pl.ds(0, 16)] * saz + gyz[k, i, pl.ds(16, 16)] * sbz) * say
                  + (gyz[k, i, pl.ds(32, 16)] * saz + gyz[k, i, pl.ds(48, 16)] * sbz) * sby)
            l0 = glx[k, i, pl.ds(0, 16)] * sax + glx[k, i, pl.ds(16, 16)] * sbx
            fv[k, i, pl.ds(0, 16)] = l0 * p0

            p1 = ((gzx[k, i, pl.ds(0, 16)] * sax + gzx[k, i, pl.ds(16, 16)] * sbx) * saz
                  + (gzx[k, i, pl.ds(32, 16)] * sax + gzx[k, i, pl.ds(48, 16)] * sbx) * sbz)
            l1 = gly[k, i, pl.ds(0, 16)] * say + gly[k, i, pl.ds(16, 16)] * sby
            fv[k, i, pl.ds(16, 16)] = l1 * p1

            p2 = ((gxy[k, i, pl.ds(0, 16)] * say + gxy[k, i, pl.ds(16, 16)] * sby) * sax
                  + (gxy[k, i, pl.ds(32, 16)] * say + gxy[k, i, pl.ds(48, 16)] * sby) * sbx)
            l2 = glz[k, i, pl.ds(0, 16)] * saz + glz[k, i, pl.ds(16, 16)] * sbz
            fv[k, i, pl.ds(32, 16)] = l2 * p2
            return carry

        lax.fori_loop(0, _CHUNK, phase_b, 0, unroll=2)
        pltpu.async_copy(fv.at[k], feat.at[pl.ds(off, _CHUNK)], osem.at[k])

    def step(c, carry):
        k = jnp.bitwise_and(c, 1)

        @pl.when(lax.rem(c, _GPS) == 0)
        def _():
            goff = base + c * _CHUNK
            pltpu.sync_copy(xs.at[pl.ds(goff, _SUPER)], xb)
            pltpu.sync_copy(ys.at[pl.ds(goff, _SUPER)], yb)
            pltpu.sync_copy(zs.at[pl.ds(goff, _SUPER)], zb)

        issue(c, k)

        @pl.when(c > 0)
        def _():
            drain_gathers(1 - k)
            combine(c - 1, 1 - k)

        return carry

    lax.fori_loop(0, _NCHUNK, step, 0)
    last = _NCHUNK - 1
    drain_gathers(last % 2)
    combine(last, last % 2)
    pltpu.make_async_copy(fv.at[0], feat.at[pl.ds(base, _CHUNK)], osem.at[0]).wait()
    pltpu.make_async_copy(fv.at[1], feat.at[pl.ds(base, _CHUNK)], osem.at[1]).wait()


def _mm_body(f_ref, w_ref, o_ref):
    o_ref[...] = jnp.dot(f_ref[...], w_ref[...],
                         preferred_element_type=jnp.float32)


def kernel(xyz, dvx, dvy, dvz, dpyz, dpzx, dpxy, cvx, cvy, cvz, cpyz, cpzx, cpxy, W):
    qyz = _quad_plane(dpyz, cpyz)
    qzx = _quad_plane(dpzx, cpzx)
    qxy = _quad_plane(dpxy, cpxy)
    lx = _pair_line(dvx, cvx)
    ly = _pair_line(dvy, cvy)
    lz = _pair_line(dvz, cvz)
    xs = xyz[:, 0]
    ys = xyz[:, 1]
    zs = xyz[:, 2]

    mesh = plsc.VectorSubcoreMesh(core_axis_name="c", subcore_axis_name="s")
    f32 = jnp.float32
    i32 = jnp.int32
    sc = pl.kernel(
        _sc_body,
        out_type=jax.ShapeDtypeStruct((_BH, 48), f32),
        mesh=mesh,
        compiler_params=pltpu.CompilerParams(
            needs_layout_passes=False, use_tc_tiling_on_sc=False),
        scratch_types=[
            pltpu.VMEM((_SUPER,), f32), pltpu.VMEM((_SUPER,), f32), pltpu.VMEM((_SUPER,), f32),
            pltpu.VMEM((2, _CHUNK), f32), pltpu.VMEM((2, _CHUNK), f32), pltpu.VMEM((2, _CHUNK), f32),
            pltpu.VMEM((2, _CHUNK), i32), pltpu.VMEM((2, _CHUNK), i32), pltpu.VMEM((2, _CHUNK), i32),
            pltpu.VMEM((2, _CHUNK), i32), pltpu.VMEM((2, _CHUNK), i32), pltpu.VMEM((2, _CHUNK), i32),
            pltpu.VMEM((2, _CHUNK, 64), f32), pltpu.VMEM((2, _CHUNK, 64), f32), pltpu.VMEM((2, _CHUNK, 64), f32),
            pltpu.VMEM((2, _CHUNK, 32), f32), pltpu.VMEM((2, _CHUNK, 32), f32), pltpu.VMEM((2, _CHUNK, 32), f32),
            pltpu.VMEM((2, _CHUNK, 48), f32),
            pltpu.SemaphoreType.DMA((2,)),
            pltpu.SemaphoreType.DMA((2,)),
        ],
    )
    wb = jnp.zeros((48, 28), f32)
    dense_rows = jnp.concatenate(
        [jnp.arange(8), jnp.arange(8) + 16, jnp.arange(8) + 32])
    color_rows = jnp.concatenate(
        [jnp.arange(8) + 8, jnp.arange(8) + 24, jnp.arange(8) + 40])
    wb = wb.at[dense_rows, 0].set(1.0)
    wb = wb.at[color_rows, 1:28].set(W.T)

    tm = 2048
    mm = pl.pallas_call(
        _mm_body,
        grid=(_BH // tm,),
        in_specs=[
            pl.BlockSpec((tm, 48), lambda i: (i, 0)),
            pl.BlockSpec((48, 28), lambda i: (0, 0)),
        ],
        out_specs=pl.BlockSpec((tm, 28), lambda i: (i, 0)),
        out_shape=jax.ShapeDtypeStruct((_BH, 28), f32),
    )

    parts = []
    for h in range(_NSPLIT):
        s = slice(h * _BH, (h + 1) * _BH)
        feat = sc(xs[s], ys[s], zs[s], qyz, qzx, qxy, lx, ly, lz)
        parts.append(mm(feat, wb))
    return jnp.concatenate(parts, axis=0)
